# baseline (speedup 1.0000x reference)
import functools

import jax
import jax.numpy as jnp
from jax import lax
from jax.experimental import pallas as pl
from jax.experimental.pallas import tpu as pltpu
from jax.experimental.pallas import tpu_sc as plsc

_VOCAB = 100000
_EMBED = 128
_BATCH = 1024
_CTX = 50

_NC = 2
_NS = 16
_LANES = 16
_NW = _NC * _NS
_B_PER_W = _BATCH // _NW
_EV = _EMBED // _LANES


def _sc_pool_body(emb_hbm, idx_hbm, out_hbm, idx_v, rows_v, pool_v, sem):
    wid = lax.axis_index("s") * _NC + lax.axis_index("c")
    base = wid * _B_PER_W
    pltpu.sync_copy(idx_hbm.at[pl.ds(base, _B_PER_W)], idx_v)

    def do_row(b, carry):
        pltpu.async_copy(emb_hbm.at[idx_v.at[b]], rows_v, sem).wait()
        scale = 1.0 / _CTX
        for j in range(_EV):
            acc = rows_v[0, pl.ds(j * _LANES, _LANES)]
            for c in range(1, _CTX):
                acc = acc + rows_v[c, pl.ds(j * _LANES, _LANES)]
            pool_v[b, pl.ds(j * _LANES, _LANES)] = acc * scale
        return carry

    lax.fori_loop(0, _B_PER_W, do_row, 0)
    pltpu.sync_copy(pool_v, out_hbm.at[pl.ds(base, _B_PER_W)])


@jax.jit
def _sc_pool(emb_table, idx):
    mesh = plsc.VectorSubcoreMesh(core_axis_name="c", subcore_axis_name="s")
    return pl.kernel(
        _sc_pool_body,
        out_type=jax.ShapeDtypeStruct((_BATCH, _EMBED), jnp.float32),
        mesh=mesh,
        scratch_types=[
            pltpu.VMEM((_B_PER_W, _CTX), jnp.int32),
            pltpu.VMEM((_CTX, _EMBED), jnp.float32),
            pltpu.VMEM((_B_PER_W, _EMBED), jnp.float32),
            pltpu.SemaphoreType.DMA,
        ],
    )(emb_table, idx)


_VT = 2048
_NSTEPS = _VOCAB // _VT
_TAIL = _VOCAB - _NSTEPS * _VT
_GRID = _NSTEPS + 1
_NBUF = 4
_NSPLIT = 4
_RS = _BATCH // _NSPLIT


_TAIL_A = (_TAIL // 128) * 128
_TAIL_B = _TAIL - _TAIL_A


def _matmul_body(x_ref, w_ref, b_ref, o_hbm, obuf, tail_a, tail_b, sems, tsem):
    i = pl.program_id(0)
    slot = lax.rem(i, _NBUF)

    @pl.when(i >= _NBUF)
    def _wait_prev():
        prev = i - _NBUF
        for s in range(_NSPLIT):
            pltpu.make_async_copy(
                obuf.at[slot, pl.ds(s * _RS, _RS)],
                o_hbm.at[pl.ds(s * _RS, _RS), pl.ds(prev * _VT, _VT)],
                sems.at[slot, s],
            ).wait()

    val = (
        jnp.dot(x_ref[...], w_ref[...], preferred_element_type=jnp.float32)
        + b_ref[...]
    )

    @pl.when(i < _NSTEPS)
    def _store_full():
        obuf[slot] = val
        for s in range(_NSPLIT):
            pltpu.make_async_copy(
                obuf.at[slot, pl.ds(s * _RS, _RS)],
                o_hbm.at[pl.ds(s * _RS, _RS), pl.ds(i * _VT, _VT)],
                sems.at[slot, s],
            ).start()

    @pl.when(i == _NSTEPS)
    def _store_tail_and_drain():
        tail_a[...] = val[:, :_TAIL_A]
        tail_b[...] = val[:, _TAIL_A:_TAIL]
        for s in range(_NSPLIT):
            pltpu.make_async_copy(
                tail_a.at[pl.ds(s * _RS, _RS)],
                o_hbm.at[pl.ds(s * _RS, _RS), pl.ds(_NSTEPS * _VT, _TAIL_A)],
                sems.at[slot, s],
            ).start()
        pltpu.make_async_copy(
            tail_b,
            o_hbm.at[:, pl.ds(_NSTEPS * _VT + _TAIL_A, _TAIL_B)],
            tsem,
        ).start()
        for back in range(_NBUF - 1, 0, -1):
            step = _NSTEPS - back
            sl = step % _NBUF
            for s in range(_NSPLIT):
                pltpu.make_async_copy(
                    obuf.at[sl, pl.ds(s * _RS, _RS)],
                    o_hbm.at[pl.ds(s * _RS, _RS), pl.ds(step * _VT, _VT)],
                    sems.at[sl, s],
                ).wait()
        for s in range(_NSPLIT):
            pltpu.make_async_copy(
                tail_a.at[pl.ds(s * _RS, _RS)],
                o_hbm.at[pl.ds(s * _RS, _RS), pl.ds(_NSTEPS * _VT, _TAIL_A)],
                sems.at[slot, s],
            ).wait()
        pltpu.make_async_copy(
            tail_b,
            o_hbm.at[:, pl.ds(_NSTEPS * _VT + _TAIL_A, _TAIL_B)],
            tsem,
        ).wait()


@jax.jit
def _project(pooled, dense_w, dense_b):
    return pl.pallas_call(
        _matmul_body,
        grid=(_GRID,),
        in_specs=[
            pl.BlockSpec((_BATCH, _EMBED), lambda i: (0, 0)),
            pl.BlockSpec((_EMBED, _VT), lambda i: (0, i)),
            pl.BlockSpec((1, _VT), lambda i: (0, i)),
        ],
        out_specs=pl.BlockSpec(memory_space=pl.ANY),
        out_shape=jax.ShapeDtypeStruct((_BATCH, _VOCAB), jnp.float32),
        scratch_shapes=[
            pltpu.VMEM((_NBUF, _BATCH, _VT), jnp.float32),
            pltpu.VMEM((_BATCH, _TAIL_A), jnp.float32),
            pltpu.VMEM((_BATCH, _TAIL_B), jnp.float32),
            pltpu.SemaphoreType.DMA((_NBUF, _NSPLIT)),
            pltpu.SemaphoreType.DMA,
        ],
        compiler_params=pltpu.CompilerParams(
            dimension_semantics=("arbitrary",),
        ),
    )(pooled, dense_w, dense_b.reshape(1, _VOCAB))


def kernel(inputs, emb_table, dense_w, dense_b):
    idx = inputs.astype(jnp.int32)
    pooled = _sc_pool(emb_table, idx)
    return _project(pooled, dense_w, dense_b)


# speedup vs baseline: 2.7292x; 2.7292x over previous
import functools

import jax
import jax.numpy as jnp
from jax import lax
from jax.experimental import pallas as pl
from jax.experimental.pallas import tpu as pltpu
from jax.experimental.pallas import tpu_sc as plsc

_VOCAB = 100000
_EMBED = 128
_BATCH = 1024
_CTX = 50

_NC = 2
_NS = 16
_LANES = 16
_NW = _NC * _NS
_B_PER_W = _BATCH // _NW
_EV = _EMBED // _LANES


def _sc_pool_body(emb_hbm, idx_hbm, out_hbm, idx_v, rows_v, pool_v, sem):
    wid = lax.axis_index("s") * _NC + lax.axis_index("c")
    base = wid * _B_PER_W
    pltpu.sync_copy(idx_hbm.at[pl.ds(base, _B_PER_W)], idx_v)

    def do_row(b, carry):
        pltpu.async_copy(emb_hbm.at[idx_v.at[b]], rows_v, sem).wait()
        scale = 1.0 / _CTX
        for j in range(_EV):
            acc = rows_v[0, pl.ds(j * _LANES, _LANES)]
            for c in range(1, _CTX):
                acc = acc + rows_v[c, pl.ds(j * _LANES, _LANES)]
            pool_v[b, pl.ds(j * _LANES, _LANES)] = acc * scale
        return carry

    lax.fori_loop(0, _B_PER_W, do_row, 0)
    pltpu.sync_copy(pool_v, out_hbm.at[pl.ds(base, _B_PER_W)])


@jax.jit
def _sc_pool(emb_table, idx):
    mesh = plsc.VectorSubcoreMesh(core_axis_name="c", subcore_axis_name="s")
    return pl.kernel(
        _sc_pool_body,
        out_type=jax.ShapeDtypeStruct((_BATCH, _EMBED), jnp.float32),
        mesh=mesh,
        scratch_types=[
            pltpu.VMEM((_B_PER_W, _CTX), jnp.int32),
            pltpu.VMEM((_CTX, _EMBED), jnp.float32),
            pltpu.VMEM((_B_PER_W, _EMBED), jnp.float32),
            pltpu.SemaphoreType.DMA,
        ],
    )(emb_table, idx)


_VT = 2000


def _matmul_body(wt_ref, x_ref, b_ref, o_ref):
    acc = jax.lax.dot_general(
        wt_ref[...],
        x_ref[...],
        (((1,), (1,)), ((), ())),
        preferred_element_type=jnp.float32,
    )
    o_ref[...] = acc + jnp.reshape(b_ref[...], (_VT, 1))


@jax.jit
def _project(pooled, dense_w, dense_b):
    wt = dense_w.T
    out_t = pl.pallas_call(
        _matmul_body,
        grid=(_VOCAB // _VT,),
        in_specs=[
            pl.BlockSpec((_VT, _EMBED), lambda i: (i, 0)),
            pl.BlockSpec((_BATCH, _EMBED), lambda i: (0, 0)),
            pl.BlockSpec((1, 1, _VT), lambda i: (i, 0, 0)),
        ],
        out_specs=pl.BlockSpec((_VT, _BATCH), lambda i: (i, 0)),
        out_shape=jax.ShapeDtypeStruct((_VOCAB, _BATCH), jnp.float32),
    )(wt, pooled, dense_b.reshape(_VOCAB // _VT, 1, _VT))
    return out_t.T


def kernel(inputs, emb_table, dense_w, dense_b):
    idx = inputs.astype(jnp.int32)
    pooled = _sc_pool(emb_table, idx)
    return _project(pooled, dense_w, dense_b)


# speedup vs baseline: 3.1395x; 1.1503x over previous
import functools

import jax
import jax.numpy as jnp
from jax import lax
from jax.experimental import pallas as pl
from jax.experimental.pallas import tpu as pltpu
from jax.experimental.pallas import tpu_sc as plsc

_VOCAB = 100000
_EMBED = 128
_BATCH = 1024
_CTX = 50

_NC = 2
_NS = 16
_LANES = 16
_NW = _NC * _NS
_B_PER_W = _BATCH // _NW
_EV = _EMBED // _LANES


_RB = 4


def _sc_pool_body(emb_hbm, idx_hbm, out_hbm, idx_v, rows_v, pool_v, sems):
    wid = lax.axis_index("s") * _NC + lax.axis_index("c")
    base = wid * _B_PER_W
    pltpu.sync_copy(idx_hbm.at[pl.ds(base, _B_PER_W)], idx_v)

    def gather(b, k):
        return pltpu.make_async_copy(
            emb_hbm.at[idx_v.at[b]], rows_v.at[k], sems.at[k]
        )

    for k in range(_RB - 1):
        gather(k, k).start()

    def do_row(b, carry):
        k = lax.rem(b, _RB)
        gather(b, k).wait()
        nb = b + _RB - 1

        @pl.when(nb < _B_PER_W)
        def _prefetch():
            gather(nb, lax.rem(nb, _RB)).start()

        scale = 1.0 / _CTX
        for j in range(_EV):
            acc = rows_v[k, 0, pl.ds(j * _LANES, _LANES)]
            for c in range(1, _CTX):
                acc = acc + rows_v[k, c, pl.ds(j * _LANES, _LANES)]
            pool_v[b, pl.ds(j * _LANES, _LANES)] = acc * scale
        return carry

    lax.fori_loop(0, _B_PER_W, do_row, 0)
    pltpu.sync_copy(pool_v, out_hbm.at[pl.ds(base, _B_PER_W)])


@jax.jit
def _sc_pool(emb_table, idx):
    mesh = plsc.VectorSubcoreMesh(core_axis_name="c", subcore_axis_name="s")
    return pl.kernel(
        _sc_pool_body,
        out_type=jax.ShapeDtypeStruct((_BATCH, _EMBED), jnp.float32),
        mesh=mesh,
        scratch_types=[
            pltpu.VMEM((_B_PER_W, _CTX), jnp.int32),
            pltpu.VMEM((_RB, _CTX, _EMBED), jnp.float32),
            pltpu.VMEM((_B_PER_W, _EMBED), jnp.float32),
            pltpu.SemaphoreType.DMA((_RB,)),
        ],
    )(emb_table, idx)


_VT = 2000


def _matmul_body(wt_ref, x_ref, b_ref, o_ref):
    acc = jax.lax.dot_general(
        wt_ref[...],
        x_ref[...],
        (((1,), (1,)), ((), ())),
        preferred_element_type=jnp.float32,
    )
    o_ref[...] = acc + jnp.reshape(b_ref[...], (_VT, 1))


@jax.jit
def _project(pooled, dense_w, dense_b):
    wt = dense_w.T
    out_t = pl.pallas_call(
        _matmul_body,
        grid=(_VOCAB // _VT,),
        in_specs=[
            pl.BlockSpec((_VT, _EMBED), lambda i: (i, 0)),
            pl.BlockSpec((_BATCH, _EMBED), lambda i: (0, 0)),
            pl.BlockSpec((1, 1, _VT), lambda i: (i, 0, 0)),
        ],
        out_specs=pl.BlockSpec((_VT, _BATCH), lambda i: (i, 0)),
        out_shape=jax.ShapeDtypeStruct((_VOCAB, _BATCH), jnp.float32),
    )(wt, pooled, dense_b.reshape(_VOCAB // _VT, 1, _VT))
    return out_t.T


def kernel(inputs, emb_table, dense_w, dense_b):
    idx = inputs.astype(jnp.int32)
    pooled = _sc_pool(emb_table, idx)
    return _project(pooled, dense_w, dense_b)


# speedup vs baseline: 3.1904x; 1.0162x over previous
import functools

import jax
import jax.numpy as jnp
from jax import lax
from jax.experimental import pallas as pl
from jax.experimental.pallas import tpu as pltpu
from jax.experimental.pallas import tpu_sc as plsc

_VOCAB = 100000
_EMBED = 128
_BATCH = 1024
_CTX = 50

_NC = 2
_NS = 16
_LANES = 16
_NW = _NC * _NS
_B_PER_W = _BATCH // _NW
_EV = _EMBED // _LANES


_RB = 4


def _sc_pool_body(emb_hbm, idx_hbm, out_hbm, idx_v, rows_v, pool_v, sems):
    wid = lax.axis_index("s") * _NC + lax.axis_index("c")
    base = wid * _B_PER_W
    pltpu.sync_copy(idx_hbm.at[pl.ds(base, _B_PER_W)], idx_v)

    def gather(b, k):
        return pltpu.make_async_copy(
            emb_hbm.at[idx_v.at[b]], rows_v.at[k], sems.at[k]
        )

    for k in range(_RB - 1):
        gather(k, k).start()

    def do_row(b, carry):
        k = lax.rem(b, _RB)
        gather(b, k).wait()
        nb = b + _RB - 1

        @pl.when(nb < _B_PER_W)
        def _prefetch():
            gather(nb, lax.rem(nb, _RB)).start()

        scale = 1.0 / _CTX
        for j in range(_EV):
            acc = rows_v[k, 0, pl.ds(j * _LANES, _LANES)]
            for c in range(1, _CTX):
                acc = acc + rows_v[k, c, pl.ds(j * _LANES, _LANES)]
            pool_v[b, pl.ds(j * _LANES, _LANES)] = acc * scale
        return carry

    lax.fori_loop(0, _B_PER_W, do_row, 0)
    pltpu.sync_copy(pool_v, out_hbm.at[pl.ds(base, _B_PER_W)])


@jax.jit
def _sc_pool(emb_table, idx):
    mesh = plsc.VectorSubcoreMesh(core_axis_name="c", subcore_axis_name="s")
    return pl.kernel(
        _sc_pool_body,
        out_type=jax.ShapeDtypeStruct((_BATCH, _EMBED), jnp.float32),
        mesh=mesh,
        scratch_types=[
            pltpu.VMEM((_B_PER_W, _CTX), jnp.int32),
            pltpu.VMEM((_RB, _CTX, _EMBED), jnp.float32),
            pltpu.VMEM((_B_PER_W, _EMBED), jnp.float32),
            pltpu.SemaphoreType.DMA((_RB,)),
        ],
    )(emb_table, idx)


_VT = 4000


def _matmul_body(wt_ref, x_ref, b_ref, o_ref):
    acc = jax.lax.dot_general(
        wt_ref[...],
        x_ref[...],
        (((1,), (1,)), ((), ())),
        preferred_element_type=jnp.float32,
    )
    o_ref[...] = acc + jnp.reshape(b_ref[...], (_VT, 1))


@jax.jit
def _project(pooled, dense_w, dense_b):
    wt = dense_w.T
    out_t = pl.pallas_call(
        _matmul_body,
        grid=(_VOCAB // _VT,),
        in_specs=[
            pl.BlockSpec((_VT, _EMBED), lambda i: (i, 0)),
            pl.BlockSpec((_BATCH, _EMBED), lambda i: (0, 0)),
            pl.BlockSpec((1, 1, _VT), lambda i: (i, 0, 0)),
        ],
        out_specs=pl.BlockSpec((_VT, _BATCH), lambda i: (i, 0)),
        out_shape=jax.ShapeDtypeStruct((_VOCAB, _BATCH), jnp.float32),
    )(wt, pooled, dense_b.reshape(_VOCAB // _VT, 1, _VT))
    return out_t.T


def kernel(inputs, emb_table, dense_w, dense_b):
    idx = inputs.astype(jnp.int32)
    pooled = _sc_pool(emb_table, idx)
    return _project(pooled, dense_w, dense_b)


# speedup vs baseline: 3.1983x; 1.0025x over previous
import functools

import jax
import jax.numpy as jnp
from jax import lax
from jax.experimental import pallas as pl
from jax.experimental.pallas import tpu as pltpu
from jax.experimental.pallas import tpu_sc as plsc

_VOCAB = 100000
_EMBED = 128
_BATCH = 1024
_CTX = 50

_NC = 2
_NS = 16
_LANES = 16
_NW = _NC * _NS
_B_PER_W = _BATCH // _NW
_EV = _EMBED // _LANES


_RB = 4


def _sc_pool_body(emb_hbm, idx_hbm, out_hbm, idx_v, rows_v, pool_v, sems):
    wid = lax.axis_index("s") * _NC + lax.axis_index("c")
    base = wid * _B_PER_W
    pltpu.sync_copy(idx_hbm.at[pl.ds(base, _B_PER_W)], idx_v)

    def gather(b, k):
        return pltpu.make_async_copy(
            emb_hbm.at[idx_v.at[b]], rows_v.at[k], sems.at[k]
        )

    for k in range(_RB - 1):
        gather(k, k).start()

    def do_row(b, carry):
        k = lax.rem(b, _RB)
        gather(b, k).wait()
        nb = b + _RB - 1

        @pl.when(nb < _B_PER_W)
        def _prefetch():
            gather(nb, lax.rem(nb, _RB)).start()

        scale = 1.0 / _CTX
        for j in range(_EV):
            acc = rows_v[k, 0, pl.ds(j * _LANES, _LANES)]
            for c in range(1, _CTX):
                acc = acc + rows_v[k, c, pl.ds(j * _LANES, _LANES)]
            pool_v[b, pl.ds(j * _LANES, _LANES)] = acc * scale
        return carry

    lax.fori_loop(0, _B_PER_W, do_row, 0)
    pltpu.sync_copy(pool_v, out_hbm.at[pl.ds(base, _B_PER_W)])


@jax.jit
def _sc_pool(emb_table, idx):
    mesh = plsc.VectorSubcoreMesh(core_axis_name="c", subcore_axis_name="s")
    return pl.kernel(
        _sc_pool_body,
        out_type=jax.ShapeDtypeStruct((_BATCH, _EMBED), jnp.float32),
        mesh=mesh,
        scratch_types=[
            pltpu.VMEM((_B_PER_W, _CTX), jnp.int32),
            pltpu.VMEM((_RB, _CTX, _EMBED), jnp.float32),
            pltpu.VMEM((_B_PER_W, _EMBED), jnp.float32),
            pltpu.SemaphoreType.DMA((_RB,)),
        ],
    )(emb_table, idx)


_VT = 5000


def _matmul_body(wt_ref, x_ref, b_ref, o_ref):
    acc = jax.lax.dot_general(
        wt_ref[...],
        x_ref[...],
        (((1,), (1,)), ((), ())),
        preferred_element_type=jnp.float32,
    )
    o_ref[...] = acc + jnp.reshape(b_ref[...], (_VT, 1))


@jax.jit
def _project(pooled, dense_w, dense_b):
    wt = dense_w.T
    out_t = pl.pallas_call(
        _matmul_body,
        grid=(_VOCAB // _VT,),
        in_specs=[
            pl.BlockSpec((_VT, _EMBED), lambda i: (i, 0)),
            pl.BlockSpec((_BATCH, _EMBED), lambda i: (0, 0)),
            pl.BlockSpec((1, 1, _VT), lambda i: (i, 0, 0)),
        ],
        out_specs=pl.BlockSpec((_VT, _BATCH), lambda i: (i, 0)),
        out_shape=jax.ShapeDtypeStruct((_VOCAB, _BATCH), jnp.float32),
    )(wt, pooled, dense_b.reshape(_VOCAB // _VT, 1, _VT))
    return out_t.T


def kernel(inputs, emb_table, dense_w, dense_b):
    idx = inputs.astype(jnp.int32)
    pooled = _sc_pool(emb_table, idx)
    return _project(pooled, dense_w, dense_b)
